# baseline (device time: 124513 ns/iter reference)
import jax
import jax.numpy as jnp
from jax import lax
from jax.experimental import pallas as pl
from jax.experimental.pallas import tpu as pltpu

N_DEV = 16
B, SQ, SKV, DH = 2, 128, 128, 64
H_PER = 4
D_MODEL = 512


def kernel(x, Wq, K_ext, V_ext, Wo):
    my = lax.axis_index("i")
    Ks = lax.dynamic_slice_in_dim(K_ext, my * H_PER, H_PER, axis=2)
    Vs = lax.dynamic_slice_in_dim(V_ext, my * H_PER, H_PER, axis=2)
    Ks = jnp.transpose(Ks, (0, 2, 1, 3))
    Vs = jnp.transpose(Vs, (0, 2, 1, 3))

    def body(x_ref, wq_ref, k_ref, v_ref, wo_ref, out_ref,
             comm_ref, send_sems, recv_sems):
        my_pos = lax.axis_index("i")
        left = lax.rem(my_pos + (N_DEV - 1), N_DEV)
        right = lax.rem(my_pos + 1, N_DEV)

        barrier_sem = pltpu.get_barrier_semaphore()
        for nbr in (left, right):
            pl.semaphore_signal(
                barrier_sem, inc=1,
                device_id=(nbr,), device_id_type=pl.DeviceIdType.MESH,
            )
        pl.semaphore_wait(barrier_sem, 2)

        wq = wq_ref[...].astype(jnp.bfloat16)
        wo = wo_ref[...].astype(jnp.bfloat16)
        for b in range(B):
            xb = x_ref[b].astype(jnp.bfloat16)
            q = jnp.dot(xb, wq, preferred_element_type=jnp.float32)
            ctx_parts = []
            for h in range(H_PER):
                qh = q[:, h * DH:(h + 1) * DH].astype(jnp.bfloat16)
                kh = k_ref[b, h].astype(jnp.bfloat16)
                vh = v_ref[b, h].astype(jnp.bfloat16)
                s = lax.dot_general(
                    qh, kh, (((1,), (1,)), ((), ())),
                    preferred_element_type=jnp.float32,
                ) * 0.125
                s = s - jnp.max(s, axis=-1, keepdims=True)
                w = jnp.exp(s)
                w = w / jnp.sum(w, axis=-1, keepdims=True)
                ctx_parts.append(jnp.dot(
                    w.astype(jnp.bfloat16), vh,
                    preferred_element_type=jnp.float32,
                ))
            ctx = jnp.concatenate(ctx_parts, axis=1).astype(jnp.bfloat16)
            part = jnp.dot(ctx, wo, preferred_element_type=jnp.float32)
            out_ref[b] = part
            comm_ref[0, b] = part

        for hop in range(1, N_DEV):
            rdma = pltpu.make_async_remote_copy(
                src_ref=comm_ref.at[hop - 1],
                dst_ref=comm_ref.at[hop],
                send_sem=send_sems.at[hop],
                recv_sem=recv_sems.at[hop],
                device_id=(right,),
                device_id_type=pl.DeviceIdType.MESH,
            )
            rdma.start()
            rdma.wait()
            out_ref[...] = out_ref[...] + comm_ref[hop]

    return pl.pallas_call(
        body,
        out_shape=jax.ShapeDtypeStruct((B, SQ, D_MODEL), jnp.float32),
        in_specs=[pl.BlockSpec(memory_space=pltpu.VMEM)] * 5,
        out_specs=pl.BlockSpec(memory_space=pltpu.VMEM),
        scratch_shapes=[
            pltpu.VMEM((N_DEV, B, SQ, D_MODEL), jnp.float32),
            pltpu.SemaphoreType.DMA((N_DEV,)),
            pltpu.SemaphoreType.DMA((N_DEV,)),
        ],
        compiler_params=pltpu.CompilerParams(collective_id=0),
    )(x, Wq, Ks, Vs, Wo)


# device time: 20996 ns/iter; 5.9303x vs baseline; 5.9303x over previous
import jax
import jax.numpy as jnp
from jax import lax
from jax.experimental import pallas as pl
from jax.experimental.pallas import tpu as pltpu

N_DEV = 16
B, SQ, SKV, DH = 2, 128, 128, 64
H_PER = 4
D_MODEL = 512
ROWS = (B * SQ) // N_DEV


def kernel(x, Wq, K_ext, V_ext, Wo):
    my = lax.axis_index("i")
    Ks = lax.dynamic_slice_in_dim(K_ext, my * H_PER, H_PER, axis=2)
    Vs = lax.dynamic_slice_in_dim(V_ext, my * H_PER, H_PER, axis=2)
    Ks = jnp.transpose(Ks, (0, 2, 1, 3))
    Vs = jnp.transpose(Vs, (0, 2, 1, 3))

    def body(x_ref, wq_ref, k_ref, v_ref, wo_ref, out_ref,
             send_buf, recv1, gbuf,
             send_sems1, recv_sems1, send_sems2, recv_sems2):
        my_pos = lax.axis_index("i")

        barrier_sem = pltpu.get_barrier_semaphore()
        for off in range(1, N_DEV):
            peer = lax.rem(my_pos + off, N_DEV)
            pl.semaphore_signal(
                barrier_sem, inc=1,
                device_id=(peer,), device_id_type=pl.DeviceIdType.MESH,
            )
        pl.semaphore_wait(barrier_sem, N_DEV - 1)

        wq = wq_ref[...].astype(jnp.bfloat16)
        wo = wo_ref[...].astype(jnp.bfloat16)
        for b in range(B):
            xb = x_ref[b].astype(jnp.bfloat16)
            q = jnp.dot(xb, wq, preferred_element_type=jnp.float32)
            ctx_parts = []
            for h in range(H_PER):
                qh = q[:, h * DH:(h + 1) * DH].astype(jnp.bfloat16)
                kh = k_ref[b, h].astype(jnp.bfloat16)
                vh = v_ref[b, h].astype(jnp.bfloat16)
                s = lax.dot_general(
                    qh, kh, (((1,), (1,)), ((), ())),
                    preferred_element_type=jnp.float32,
                ) * 0.125
                s = s - jnp.max(s, axis=-1, keepdims=True)
                w = jnp.exp(s)
                w = w / jnp.sum(w, axis=-1, keepdims=True)
                ctx_parts.append(jnp.dot(
                    w.astype(jnp.bfloat16), vh,
                    preferred_element_type=jnp.float32,
                ))
            ctx = jnp.concatenate(ctx_parts, axis=1).astype(jnp.bfloat16)
            part = jnp.dot(ctx, wo, preferred_element_type=jnp.float32)
            send_buf[b * SQ:(b + 1) * SQ, :] = part.astype(jnp.bfloat16)

        sends1 = []
        for off in range(1, N_DEV):
            p = lax.rem(my_pos + off, N_DEV)
            rdma = pltpu.make_async_remote_copy(
                src_ref=send_buf.at[pl.ds(p * ROWS, ROWS)],
                dst_ref=recv1.at[my_pos],
                send_sem=send_sems1.at[off],
                recv_sem=recv_sems1.at[my_pos],
                device_id=(p,),
                device_id_type=pl.DeviceIdType.MESH,
            )
            rdma.start()
            sends1.append(rdma)

        acc = send_buf[pl.ds(my_pos * ROWS, ROWS), :].astype(jnp.float32)
        for off in range(1, N_DEV):
            src = lax.rem(my_pos + off, N_DEV)
            recv = pltpu.make_async_remote_copy(
                src_ref=send_buf.at[pl.ds(0, ROWS)],
                dst_ref=recv1.at[src],
                send_sem=send_sems1.at[off],
                recv_sem=recv_sems1.at[src],
                device_id=(src,),
                device_id_type=pl.DeviceIdType.MESH,
            )
            recv.wait_recv()
            acc = acc + recv1[src].astype(jnp.float32)

        gbuf[pl.ds(my_pos * ROWS, ROWS), :] = acc.astype(jnp.bfloat16)
        sends2 = []
        for off in range(1, N_DEV):
            p = lax.rem(my_pos + off, N_DEV)
            rdma = pltpu.make_async_remote_copy(
                src_ref=gbuf.at[pl.ds(my_pos * ROWS, ROWS)],
                dst_ref=gbuf.at[pl.ds(my_pos * ROWS, ROWS)],
                send_sem=send_sems2.at[off],
                recv_sem=recv_sems2.at[my_pos],
                device_id=(p,),
                device_id_type=pl.DeviceIdType.MESH,
            )
            rdma.start()
            sends2.append(rdma)

        for off in range(1, N_DEV):
            src = lax.rem(my_pos + off, N_DEV)
            recv = pltpu.make_async_remote_copy(
                src_ref=gbuf.at[pl.ds(0, ROWS)],
                dst_ref=gbuf.at[pl.ds(src * ROWS, ROWS)],
                send_sem=send_sems2.at[off],
                recv_sem=recv_sems2.at[src],
                device_id=(src,),
                device_id_type=pl.DeviceIdType.MESH,
            )
            recv.wait_recv()

        for rdma in sends1 + sends2:
            rdma.wait_send()

        out_ref[...] = gbuf[...].astype(jnp.float32).reshape(B, SQ, D_MODEL)

    return pl.pallas_call(
        body,
        out_shape=jax.ShapeDtypeStruct((B, SQ, D_MODEL), jnp.float32),
        in_specs=[pl.BlockSpec(memory_space=pltpu.VMEM)] * 5,
        out_specs=pl.BlockSpec(memory_space=pltpu.VMEM),
        scratch_shapes=[
            pltpu.VMEM((B * SQ, D_MODEL), jnp.bfloat16),
            pltpu.VMEM((N_DEV, ROWS, D_MODEL), jnp.bfloat16),
            pltpu.VMEM((B * SQ, D_MODEL), jnp.bfloat16),
            pltpu.SemaphoreType.DMA((N_DEV,)),
            pltpu.SemaphoreType.DMA((N_DEV,)),
            pltpu.SemaphoreType.DMA((N_DEV,)),
            pltpu.SemaphoreType.DMA((N_DEV,)),
        ],
        compiler_params=pltpu.CompilerParams(collective_id=0),
    )(x, Wq, Ks, Vs, Wo)
